# Initial kernel scaffold; baseline (speedup 1.0000x reference)
import functools

import jax
import jax.numpy as jnp
from jax import lax
from jax.experimental import pallas as pl
from jax.experimental.pallas import tpu as pltpu
from jax.experimental.pallas import tpu_sc as plsc

L = 16
NTILES = 16
NCORES = 2
NW = NTILES * NCORES

_sc_params = pltpu.CompilerParams(needs_layout_passes=False)
_mesh = plsc.VectorSubcoreMesh(core_axis_name="c", subcore_axis_name="s")


def _zero_table(tbl, np_):
    zeros = jnp.zeros((L,), jnp.float32)

    def body(i, carry):
        tbl[pl.ds(i * L, L)] = zeros
        return carry

    lax.fori_loop(0, np_ // L, body, 0)


def _reduce_partials(T, b2, shared, out_hbm, cid, sid, np_):
    seg = np_ // NTILES
    pltpu.sync_copy(T, shared.at[sid])
    plsc.subcore_barrier()
    for t in range(NTILES):
        pltpu.sync_copy(shared.at[t, pl.ds(sid * seg, seg)],
                        T.at[pl.ds(t * seg, seg)])

    def body(j, carry):
        acc = T[pl.ds(j * L, L)]
        for t in range(1, NTILES):
            acc = acc + T[pl.ds(t * seg + j * L, L)]
        b2[pl.ds(j * L, L)] = acc
        return carry

    lax.fori_loop(0, seg // L, body, 0)
    pltpu.sync_copy(b2.at[pl.ds(0, seg)], out_hbm.at[cid, pl.ds(sid * seg, seg)])


def _scatter_phase(idx_hbm, val_hbm, T, b1, b2, base, epw, chunk):
    def chunk_body(c, carry):
        off = base + c * chunk
        pltpu.sync_copy(idx_hbm.at[pl.ds(off, chunk)], b1.at[pl.ds(0, chunk)])
        pltpu.sync_copy(val_hbm.at[pl.ds(off, chunk)], b2.at[pl.ds(0, chunk)])

        def vec_body(j, inner):
            iv = b1[pl.ds(j * L, L)]
            vv = b2[pl.ds(j * L, L)]
            plsc.addupdate_scatter(T, [iv], vv)
            return inner

        lax.fori_loop(0, chunk // L, vec_body, 0)
        return carry

    lax.fori_loop(0, epw // chunk, chunk_body, 0)


def _k1_body(np_, epw, chunk, var0_hbm, cidx_hbm, vidx_hbm, attr_hbm,
             aggp_hbm, prod_hbm, T, b1, b2, b3, shared):
    cid = lax.axis_index("c")
    sid = lax.axis_index("s")
    wid = sid * NCORES + cid
    base = wid * epw

    pltpu.sync_copy(var0_hbm, T)

    def chunk_a(c, carry):
        off = base + c * chunk
        pltpu.sync_copy(vidx_hbm.at[pl.ds(off, chunk)], b1.at[pl.ds(0, chunk)])
        pltpu.sync_copy(attr_hbm.at[pl.ds(off, chunk)], b2.at[pl.ds(0, chunk)])

        def vec_a(j, inner):
            iv = b1[pl.ds(j * L, L)]
            vals = plsc.load_gather(T, [iv])
            b3[pl.ds(j * L, L)] = vals * b2[pl.ds(j * L, L)]
            return inner

        lax.fori_loop(0, chunk // L, vec_a, 0)
        pltpu.sync_copy(b3.at[pl.ds(0, chunk)], prod_hbm.at[pl.ds(off, chunk)])
        return carry

    lax.fori_loop(0, epw // chunk, chunk_a, 0)

    _zero_table(T, np_)
    _scatter_phase(cidx_hbm, prod_hbm, T, b1, b2, base, epw, chunk)
    _reduce_partials(T, b2, shared, aggp_hbm, cid, sid, np_)


def _k2_body(np_, epw, chunk, aggp_hbm, con1_hbm, cidx_hbm, vidx_hbm,
             resp_hbm, prod_hbm, T, b1, b2, b3, shared):
    cid = lax.axis_index("c")
    sid = lax.axis_index("s")
    wid = sid * NCORES + cid
    base = wid * epw
    seg = np_ // NTILES

    def stage(k, carry):
        pltpu.sync_copy(aggp_hbm.at[0, pl.ds(k * seg, seg)], b2.at[pl.ds(0, seg)])
        pltpu.sync_copy(aggp_hbm.at[1, pl.ds(k * seg, seg)], b3.at[pl.ds(0, seg)])

        def add1(j, inner):
            T[pl.ds(k * seg + j * L, L)] = b2[pl.ds(j * L, L)] + b3[pl.ds(j * L, L)]
            return inner

        lax.fori_loop(0, seg // L, add1, 0)
        pltpu.sync_copy(con1_hbm.at[pl.ds(k * seg, seg)], b2.at[pl.ds(0, seg)])

        def add2(j, inner):
            d = pl.ds(k * seg + j * L, L)
            T[d] = T[d] + b2[pl.ds(j * L, L)]
            return inner

        lax.fori_loop(0, seg // L, add2, 0)
        return carry

    lax.fori_loop(0, NTILES, stage, 0)

    def chunk_c(c, carry):
        off = base + c * chunk
        pltpu.sync_copy(cidx_hbm.at[pl.ds(off, chunk)], b1.at[pl.ds(0, chunk)])

        def vec_c(j, inner):
            iv = b1[pl.ds(j * L, L)]
            b3[pl.ds(j * L, L)] = plsc.load_gather(T, [iv])
            return inner

        lax.fori_loop(0, chunk // L, vec_c, 0)
        pltpu.sync_copy(b3.at[pl.ds(0, chunk)], prod_hbm.at[pl.ds(off, chunk)])
        return carry

    lax.fori_loop(0, epw // chunk, chunk_c, 0)

    _zero_table(T, np_)
    _scatter_phase(vidx_hbm, prod_hbm, T, b1, b2, base, epw, chunk)
    _reduce_partials(T, b2, shared, resp_hbm, cid, sid, np_)


def _tc_finish_body(r_ref, m_ref, o_ref):
    o_ref[...] = (r_ref[0] + r_ref[1]) * m_ref[...]


@jax.jit
def kernel(constraint, variable, cv_edge_index, edge_attr, cand_mask):
    n_c = constraint.shape[0]
    n_v = variable.shape[0]
    e = edge_attr.shape[0]
    n = max(n_c, n_v)
    np_ = ((n + NTILES * L * 8 - 1) // (NTILES * L * 8)) * (NTILES * L * 8)
    epw = e // NW
    assert epw * NW == e and epw % L == 0
    chunk = 4000
    assert epw % chunk == 0 and chunk % L == 0
    seg = np_ // NTILES
    bufn = max(seg, chunk)

    c_idx = cv_edge_index[0]
    v_idx = cv_edge_index[1]
    var0 = jnp.zeros((np_,), jnp.float32).at[:n_v].set(variable[:, 0])
    con1 = jnp.zeros((np_,), jnp.float32).at[:n_c].set(constraint[:, 1])
    maskf = jnp.zeros((np_,), jnp.float32).at[:n_v].set(cand_mask.astype(jnp.float32))

    sc_scratch = [
        pltpu.VMEM((np_,), jnp.float32),
        pltpu.VMEM((bufn,), jnp.int32),
        pltpu.VMEM((bufn,), jnp.float32),
        pltpu.VMEM((bufn,), jnp.float32),
        pltpu.VMEM_SHARED((NTILES, np_), jnp.float32),
    ]

    k1 = pl.kernel(
        functools.partial(_k1_body, np_, epw, chunk),
        out_type=(
            jax.ShapeDtypeStruct((NCORES, np_), jnp.float32),
            jax.ShapeDtypeStruct((e,), jnp.float32),
        ),
        mesh=_mesh,
        scratch_types=sc_scratch,
        compiler_params=_sc_params,
    )
    aggp, _prod = k1(var0, c_idx, v_idx, edge_attr)

    k2 = pl.kernel(
        functools.partial(_k2_body, np_, epw, chunk),
        out_type=(
            jax.ShapeDtypeStruct((NCORES, np_), jnp.float32),
            jax.ShapeDtypeStruct((e,), jnp.float32),
        ),
        mesh=_mesh,
        scratch_types=sc_scratch,
        compiler_params=_sc_params,
    )
    resp, _prod2 = k2(aggp, con1, c_idx, v_idx)

    out = pl.pallas_call(
        _tc_finish_body,
        out_shape=jax.ShapeDtypeStruct((np_ // 128, 128), jnp.float32),
    )(resp.reshape(NCORES, np_ // 128, 128), maskf.reshape(np_ // 128, 128))

    return out.reshape(np_)[:n_v]


# speedup vs baseline: 247.7932x; 247.7932x over previous
import functools

import jax
import jax.numpy as jnp
from jax import lax
from jax.experimental import pallas as pl
from jax.experimental.pallas import tpu as pltpu
from jax.experimental.pallas import tpu_sc as plsc

L = 16
NTILES = 16
NCORES = 2
NW = NTILES * NCORES
CHUNK = 2000

_sc_params = pltpu.CompilerParams(needs_layout_passes=False)
_mesh = plsc.VectorSubcoreMesh(core_axis_name="c", subcore_axis_name="s")


def _zero_range(tbl, n):
    zeros = jnp.zeros((L,), jnp.float32)

    def body(i, carry):
        tbl[pl.ds(i * L, L)] = zeros
        return carry

    lax.fori_loop(0, n // L, body, 0, unroll=8)


def _phase_gather(T, b1, b2, b3, idx_hbm, attr_hbm, out_hbm, base, epw, sems):
    si = sems[0:2]
    sv = sems[2:4]
    so = sems[4:6]
    nch = epw // CHUNK

    def start_in(c, s):
        off = base + c * CHUNK
        pltpu.make_async_copy(idx_hbm.at[pl.ds(off, CHUNK)],
                              b1.at[pl.ds(s * CHUNK, CHUNK)], si[s]).start()
        if attr_hbm is not None:
            pltpu.make_async_copy(attr_hbm.at[pl.ds(off, CHUNK)],
                                  b2.at[pl.ds(s * CHUNK, CHUNK)], sv[s]).start()

    def wait_in(c, s):
        off = base + c * CHUNK
        pltpu.make_async_copy(idx_hbm.at[pl.ds(off, CHUNK)],
                              b1.at[pl.ds(s * CHUNK, CHUNK)], si[s]).wait()
        if attr_hbm is not None:
            pltpu.make_async_copy(attr_hbm.at[pl.ds(off, CHUNK)],
                                  b2.at[pl.ds(s * CHUNK, CHUNK)], sv[s]).wait()

    def start_out(c, s):
        off = base + c * CHUNK
        pltpu.make_async_copy(b3.at[pl.ds(s * CHUNK, CHUNK)],
                              out_hbm.at[pl.ds(off, CHUNK)], so[s]).start()

    def wait_out(c, s):
        off = base + c * CHUNK
        pltpu.make_async_copy(b3.at[pl.ds(s * CHUNK, CHUNK)],
                              out_hbm.at[pl.ds(off, CHUNK)], so[s]).wait()

    start_in(0, 0)

    def pair(p, carry):
        for s in (0, 1):
            c = p * 2 + s

            @pl.when(c + 1 < nch)
            def _next(s=s, c=c):
                start_in(c + 1, 1 - s)

            wait_in(c, s)

            @pl.when(c >= 2)
            def _drain(s=s, c=c):
                wait_out(c - 2, s)

            if attr_hbm is not None:
                def vec(j, inner, s=s):
                    d = pl.ds(s * CHUNK + j * L, L)
                    iv = b1[d]
                    b3[d] = plsc.load_gather(T, [iv]) * b2[d]
                    return inner
            else:
                def vec(j, inner, s=s):
                    d = pl.ds(s * CHUNK + j * L, L)
                    iv = b1[d]
                    b3[d] = plsc.load_gather(T, [iv])
                    return inner

            lax.fori_loop(0, CHUNK // L, vec, 0, unroll=4)
            start_out(c, s)
        return carry

    lax.fori_loop(0, nch // 2, pair, 0)
    wait_out(nch - 2, 0)
    wait_out(nch - 1, 1)


def _phase_scatter(T, b1, b2, idx_hbm, val_hbm, base, epw, sems):
    si = sems[0:2]
    sv = sems[2:4]
    nch = epw // CHUNK

    def start_in(c, s):
        off = base + c * CHUNK
        pltpu.make_async_copy(idx_hbm.at[pl.ds(off, CHUNK)],
                              b1.at[pl.ds(s * CHUNK, CHUNK)], si[s]).start()
        pltpu.make_async_copy(val_hbm.at[pl.ds(off, CHUNK)],
                              b2.at[pl.ds(s * CHUNK, CHUNK)], sv[s]).start()

    def wait_in(c, s):
        off = base + c * CHUNK
        pltpu.make_async_copy(idx_hbm.at[pl.ds(off, CHUNK)],
                              b1.at[pl.ds(s * CHUNK, CHUNK)], si[s]).wait()
        pltpu.make_async_copy(val_hbm.at[pl.ds(off, CHUNK)],
                              b2.at[pl.ds(s * CHUNK, CHUNK)], sv[s]).wait()

    start_in(0, 0)

    def pair(p, carry):
        for s in (0, 1):
            c = p * 2 + s

            @pl.when(c + 1 < nch)
            def _next(s=s, c=c):
                start_in(c + 1, 1 - s)

            wait_in(c, s)

            def vec(j, inner, s=s):
                d = pl.ds(s * CHUNK + j * L, L)
                plsc.addupdate_scatter(T, [b1[d]], b2[d])
                return inner

            lax.fori_loop(0, CHUNK // L, vec, 0, unroll=4)
        return carry

    lax.fori_loop(0, nch // 2, pair, 0)


def _reduce_partials(T, b2, b3, part_hbm, out_hbm, con1_hbm, cid, sid, np_,
                     sems):
    seg = np_ // NTILES
    wid = sid * NCORES + cid
    cols = pl.ds(sid * seg, seg)
    si = sems[0:2]
    pltpu.sync_copy(T, part_hbm.at[wid])
    plsc.subcore_barrier()

    def start_strip(t, s):
        row = t * NCORES + cid
        pltpu.make_async_copy(part_hbm.at[row, cols],
                              b2.at[pl.ds(s * seg, seg)], si[s]).start()

    def wait_strip(t, s):
        row = t * NCORES + cid
        pltpu.make_async_copy(part_hbm.at[row, cols],
                              b2.at[pl.ds(s * seg, seg)], si[s]).wait()

    _zero_range(b3, seg)
    start_strip(0, 0)

    def rp(p, carry):
        for s in (0, 1):
            t = p * 2 + s

            @pl.when(t + 1 < NTILES)
            def _next(s=s, t=t):
                start_strip(t + 1, 1 - s)

            wait_strip(t, s)

            def ab(j, inner, s=s):
                d = pl.ds(j * L, L)
                b3[d] = b3[d] + b2[pl.ds(s * seg + j * L, L)]
                return inner

            lax.fori_loop(0, seg // L, ab, 0, unroll=4)
        return carry

    lax.fori_loop(0, NTILES // 2, rp, 0)

    if con1_hbm is not None:
        @pl.when(cid == 0)
        def _fold():
            pltpu.sync_copy(con1_hbm.at[cols], b2.at[pl.ds(0, seg)])

            def cb(j, inner):
                d = pl.ds(j * L, L)
                b3[d] = b3[d] + b2[d]
                return inner

            lax.fori_loop(0, seg // L, cb, 0, unroll=4)

    pltpu.sync_copy(b3.at[pl.ds(0, seg)], out_hbm.at[cid, cols])


def _k1_body(np_, epw, var0_hbm, con1_hbm, cidx_hbm, vidx_hbm, attr_hbm,
             aggp_hbm, prod_hbm, part_hbm, T, b1, b2, b3, *sems):
    cid = lax.axis_index("c")
    sid = lax.axis_index("s")
    base = (sid * NCORES + cid) * epw

    pltpu.sync_copy(var0_hbm, T)
    _phase_gather(T, b1, b2, b3, vidx_hbm, attr_hbm, prod_hbm, base, epw, sems)
    _zero_range(T, np_)
    _phase_scatter(T, b1, b2, cidx_hbm, prod_hbm, base, epw, sems)
    _reduce_partials(T, b2, b3, part_hbm, aggp_hbm, con1_hbm, cid, sid, np_,
                     sems)


def _k2_body(np_, epw, aggp_hbm, cidx_hbm, vidx_hbm,
             resp_hbm, prod_hbm, part_hbm, T, b1, b2, b3, *sems):
    cid = lax.axis_index("c")
    sid = lax.axis_index("s")
    base = (sid * NCORES + cid) * epw
    seg = np_ // NTILES
    si = sems[0:2]
    sv = sems[2:4]

    def start_stage(k, s):
        pltpu.make_async_copy(aggp_hbm.at[0, pl.ds(k * seg, seg)],
                              b2.at[pl.ds(s * seg, seg)], si[s]).start()
        pltpu.make_async_copy(aggp_hbm.at[1, pl.ds(k * seg, seg)],
                              b3.at[pl.ds(s * seg, seg)], sv[s]).start()

    def wait_stage(k, s):
        pltpu.make_async_copy(aggp_hbm.at[0, pl.ds(k * seg, seg)],
                              b2.at[pl.ds(s * seg, seg)], si[s]).wait()
        pltpu.make_async_copy(aggp_hbm.at[1, pl.ds(k * seg, seg)],
                              b3.at[pl.ds(s * seg, seg)], sv[s]).wait()

    start_stage(0, 0)

    def sp(p, carry):
        for s in (0, 1):
            k = p * 2 + s

            @pl.when(k + 1 < NTILES)
            def _next(s=s, k=k):
                start_stage(k + 1, 1 - s)

            wait_stage(k, s)

            def add(j, inner, s=s, k=k):
                T[pl.ds(k * seg + j * L, L)] = (
                    b2[pl.ds(s * seg + j * L, L)] + b3[pl.ds(s * seg + j * L, L)])
                return inner

            lax.fori_loop(0, seg // L, add, 0, unroll=4)
        return carry

    lax.fori_loop(0, NTILES // 2, sp, 0)

    _phase_gather(T, b1, b2, b3, cidx_hbm, None, prod_hbm, base, epw, sems)
    _zero_range(T, np_)
    _phase_scatter(T, b1, b2, vidx_hbm, prod_hbm, base, epw, sems)
    _reduce_partials(T, b2, b3, part_hbm, resp_hbm, None, cid, sid, np_, sems)


def _tc_finish_body(r_ref, m_ref, o_ref):
    o_ref[...] = (r_ref[0] + r_ref[1]) * m_ref[...]


@jax.jit
def kernel(constraint, variable, cv_edge_index, edge_attr, cand_mask):
    n_c = constraint.shape[0]
    n_v = variable.shape[0]
    e = edge_attr.shape[0]
    n = max(n_c, n_v)
    np_ = ((n + NTILES * L * 8 - 1) // (NTILES * L * 8)) * (NTILES * L * 8)
    epw = e // NW
    assert epw * NW == e and epw % (2 * CHUNK) == 0
    seg = np_ // NTILES
    bufn = max(2 * CHUNK, 2 * seg)

    c_idx = cv_edge_index[0]
    v_idx = cv_edge_index[1]
    var0 = jnp.zeros((np_,), jnp.float32).at[:n_v].set(variable[:, 0])
    con1 = jnp.zeros((np_,), jnp.float32).at[:n_c].set(constraint[:, 1])
    maskf = jnp.zeros((np_,), jnp.float32).at[:n_v].set(cand_mask.astype(jnp.float32))

    sc_scratch = [
        pltpu.VMEM((np_,), jnp.float32),
        pltpu.VMEM((2 * CHUNK,), jnp.int32),
        pltpu.VMEM((bufn,), jnp.float32),
        pltpu.VMEM((bufn,), jnp.float32),
    ] + [pltpu.SemaphoreType.DMA] * 6

    k1 = pl.kernel(
        functools.partial(_k1_body, np_, epw),
        out_type=(
            jax.ShapeDtypeStruct((NCORES, np_), jnp.float32),
            jax.ShapeDtypeStruct((e,), jnp.float32),
            jax.ShapeDtypeStruct((NW, np_), jnp.float32),
        ),
        mesh=_mesh,
        scratch_types=sc_scratch,
        compiler_params=_sc_params,
    )
    aggp, _prod, _part1 = k1(var0, con1, c_idx, v_idx, edge_attr)

    k2 = pl.kernel(
        functools.partial(_k2_body, np_, epw),
        out_type=(
            jax.ShapeDtypeStruct((NCORES, np_), jnp.float32),
            jax.ShapeDtypeStruct((e,), jnp.float32),
            jax.ShapeDtypeStruct((NW, np_), jnp.float32),
        ),
        mesh=_mesh,
        scratch_types=sc_scratch,
        compiler_params=_sc_params,
    )
    resp, _prod2, _part2 = k2(aggp, c_idx, v_idx)

    out = pl.pallas_call(
        _tc_finish_body,
        out_shape=jax.ShapeDtypeStruct((np_ // 128, 128), jnp.float32),
    )(resp.reshape(NCORES, np_ // 128, 128), maskf.reshape(np_ // 128, 128))

    return out.reshape(np_)[:n_v]


# speedup vs baseline: 500.4615x; 2.0197x over previous
import functools

import jax
import jax.numpy as jnp
from jax import lax
from jax.experimental import pallas as pl
from jax.experimental.pallas import tpu as pltpu
from jax.experimental.pallas import tpu_sc as plsc

L = 16
NTILES = 16
NCORES = 2
NW = NTILES * NCORES
CHUNK = 4000

_sc_params = pltpu.CompilerParams(needs_layout_passes=False)
_mesh = plsc.VectorSubcoreMesh(core_axis_name="c", subcore_axis_name="s")


def _ploop(n, body, unroll=8):
    plsc.parallel_loop(0, n, 1, unroll=unroll)(body)


def _zero_range(tbl, n):
    zeros = jnp.zeros((L,), jnp.float32)

    def body(i):
        tbl[pl.ds(i * L, L)] = zeros

    _ploop(n // L, body)


def _phase_gather(T, b1, b2, b3, idx_hbm, attr_hbm, out_hbm, base, epw, sems):
    si = sems[0:2]
    sv = sems[2:4]
    so = sems[4:6]
    nch = epw // CHUNK

    def start_in(c, s):
        off = base + c * CHUNK
        pltpu.make_async_copy(idx_hbm.at[pl.ds(off, CHUNK)],
                              b1.at[pl.ds(s * CHUNK, CHUNK)], si[s]).start()
        if attr_hbm is not None:
            pltpu.make_async_copy(attr_hbm.at[pl.ds(off, CHUNK)],
                                  b2.at[pl.ds(s * CHUNK, CHUNK)], sv[s]).start()

    def wait_in(c, s):
        off = base + c * CHUNK
        pltpu.make_async_copy(idx_hbm.at[pl.ds(off, CHUNK)],
                              b1.at[pl.ds(s * CHUNK, CHUNK)], si[s]).wait()
        if attr_hbm is not None:
            pltpu.make_async_copy(attr_hbm.at[pl.ds(off, CHUNK)],
                                  b2.at[pl.ds(s * CHUNK, CHUNK)], sv[s]).wait()

    def start_out(c, s):
        off = base + c * CHUNK
        pltpu.make_async_copy(b3.at[pl.ds(s * CHUNK, CHUNK)],
                              out_hbm.at[pl.ds(off, CHUNK)], so[s]).start()

    def wait_out(c, s):
        off = base + c * CHUNK
        pltpu.make_async_copy(b3.at[pl.ds(s * CHUNK, CHUNK)],
                              out_hbm.at[pl.ds(off, CHUNK)], so[s]).wait()

    start_in(0, 0)

    def pair(p, carry):
        for s in (0, 1):
            c = p * 2 + s

            @pl.when(c + 1 < nch)
            def _next(s=s, c=c):
                start_in(c + 1, 1 - s)

            wait_in(c, s)

            @pl.when(c >= 2)
            def _drain(s=s, c=c):
                wait_out(c - 2, s)

            if attr_hbm is not None:
                def vec(j, s=s):
                    d = pl.ds(s * CHUNK + j * L, L)
                    iv = b1[d]
                    b3[d] = plsc.load_gather(T, [iv]) * b2[d]
            else:
                def vec(j, s=s):
                    d = pl.ds(s * CHUNK + j * L, L)
                    iv = b1[d]
                    b3[d] = plsc.load_gather(T, [iv])

            _ploop(CHUNK // L, vec)
            start_out(c, s)
        return carry

    lax.fori_loop(0, nch // 2, pair, 0)
    if nch % 2:
        c = nch - 1

        @pl.when(c >= 2)
        def _drain_tail():
            wait_out(c - 2, 0)

        wait_in(c, 0)
        if attr_hbm is not None:
            def vect(j):
                d = pl.ds(j * L, L)
                b3[d] = plsc.load_gather(T, [b1[d]]) * b2[d]
        else:
            def vect(j):
                d = pl.ds(j * L, L)
                b3[d] = plsc.load_gather(T, [b1[d]])
        _ploop(CHUNK // L, vect)
        start_out(c, 0)
        wait_out(nch - 2, 1)
        wait_out(nch - 1, 0)
    else:
        wait_out(nch - 2, 0)
        wait_out(nch - 1, 1)


def _phase_scatter(T, b1, b2, idx_hbm, val_hbm, base, epw, sems):
    si = sems[0:2]
    sv = sems[2:4]
    nch = epw // CHUNK

    def start_in(c, s):
        off = base + c * CHUNK
        pltpu.make_async_copy(idx_hbm.at[pl.ds(off, CHUNK)],
                              b1.at[pl.ds(s * CHUNK, CHUNK)], si[s]).start()
        pltpu.make_async_copy(val_hbm.at[pl.ds(off, CHUNK)],
                              b2.at[pl.ds(s * CHUNK, CHUNK)], sv[s]).start()

    def wait_in(c, s):
        off = base + c * CHUNK
        pltpu.make_async_copy(idx_hbm.at[pl.ds(off, CHUNK)],
                              b1.at[pl.ds(s * CHUNK, CHUNK)], si[s]).wait()
        pltpu.make_async_copy(val_hbm.at[pl.ds(off, CHUNK)],
                              b2.at[pl.ds(s * CHUNK, CHUNK)], sv[s]).wait()

    start_in(0, 0)

    def pair(p, carry):
        for s in (0, 1):
            c = p * 2 + s

            @pl.when(c + 1 < nch)
            def _next(s=s, c=c):
                start_in(c + 1, 1 - s)

            wait_in(c, s)

            def vec(j, s=s):
                d = pl.ds(s * CHUNK + j * L, L)
                plsc.addupdate_scatter(T, [b1[d]], b2[d])

            _ploop(CHUNK // L, vec)
        return carry

    lax.fori_loop(0, nch // 2, pair, 0)
    if nch % 2:
        c = nch - 1
        wait_in(c, 0)

        def vect(j):
            d = pl.ds(j * L, L)
            plsc.addupdate_scatter(T, [b1[d]], b2[d])

        _ploop(CHUNK // L, vect)


def _k1_body(np_, epw, var0_hbm, cidx_hbm, vidx_hbm, attr_hbm,
             prod_hbm, part_hbm, T, b1, b2, b3, *sems):
    cid = lax.axis_index("c")
    sid = lax.axis_index("s")
    wid = sid * NCORES + cid
    base = wid * epw

    pltpu.sync_copy(var0_hbm, T)
    _phase_gather(T, b1, b2, b3, vidx_hbm, attr_hbm, prod_hbm, base, epw, sems)
    _zero_range(T, np_)
    _phase_scatter(T, b1, b2, cidx_hbm, prod_hbm, base, epw, sems)
    pltpu.sync_copy(T, part_hbm.at[wid])


def _k2_body(np_, epw, nodeval_hbm, cidx_hbm, vidx_hbm,
             prod_hbm, part_hbm, T, b1, b2, b3, *sems):
    cid = lax.axis_index("c")
    sid = lax.axis_index("s")
    wid = sid * NCORES + cid
    base = wid * epw

    pltpu.sync_copy(nodeval_hbm, T)
    _phase_gather(T, b1, b2, b3, cidx_hbm, None, prod_hbm, base, epw, sems)
    _zero_range(T, np_)
    _phase_scatter(T, b1, b2, vidx_hbm, prod_hbm, base, epw, sems)
    pltpu.sync_copy(T, part_hbm.at[wid])


def _tc_nodeval_body(p_ref, c_ref, o_ref):
    o_ref[...] = jnp.sum(p_ref[...], axis=0) + c_ref[...]


def _tc_finish_body(p_ref, m_ref, o_ref):
    o_ref[...] = jnp.sum(p_ref[...], axis=0) * m_ref[...]


def _tc_reduce(body, part, vec, np_):
    bc = np_ // 14
    return pl.pallas_call(
        body,
        grid=(14,),
        in_specs=[
            pl.BlockSpec((NW, bc), lambda i: (0, i)),
            pl.BlockSpec((bc,), lambda i: (i,)),
        ],
        out_specs=pl.BlockSpec((bc,), lambda i: (i,)),
        out_shape=jax.ShapeDtypeStruct((np_,), jnp.float32),
    )(part, vec)


@jax.jit
def kernel(constraint, variable, cv_edge_index, edge_attr, cand_mask):
    n_c = constraint.shape[0]
    n_v = variable.shape[0]
    e = edge_attr.shape[0]
    n = max(n_c, n_v)
    np_ = ((n + NTILES * L * 8 - 1) // (NTILES * L * 8)) * (NTILES * L * 8)
    epw = e // NW
    assert epw * NW == e and epw % CHUNK == 0
    bufn = 2 * CHUNK

    c_idx = cv_edge_index[0]
    v_idx = cv_edge_index[1]
    var0 = jnp.concatenate([variable[:, 0], jnp.zeros((np_ - n_v,), jnp.float32)])
    con1 = jnp.concatenate([constraint[:, 1], jnp.zeros((np_ - n_c,), jnp.float32)])
    maskf = jnp.concatenate([cand_mask.astype(jnp.float32),
                             jnp.zeros((np_ - n_v,), jnp.float32)])

    sc_scratch = [
        pltpu.VMEM((np_,), jnp.float32),
        pltpu.VMEM((2 * CHUNK,), jnp.int32),
        pltpu.VMEM((bufn,), jnp.float32),
        pltpu.VMEM((bufn,), jnp.float32),
    ] + [pltpu.SemaphoreType.DMA] * 6

    k1 = pl.kernel(
        functools.partial(_k1_body, np_, epw),
        out_type=(
            jax.ShapeDtypeStruct((e,), jnp.float32),
            jax.ShapeDtypeStruct((NW, np_), jnp.float32),
        ),
        mesh=_mesh,
        scratch_types=sc_scratch,
        compiler_params=_sc_params,
    )
    _prod, part1 = k1(var0, c_idx, v_idx, edge_attr)

    nodeval = _tc_reduce(_tc_nodeval_body, part1, con1, np_)

    k2 = pl.kernel(
        functools.partial(_k2_body, np_, epw),
        out_type=(
            jax.ShapeDtypeStruct((e,), jnp.float32),
            jax.ShapeDtypeStruct((NW, np_), jnp.float32),
        ),
        mesh=_mesh,
        scratch_types=sc_scratch,
        compiler_params=_sc_params,
    )
    _prod2, part2 = k2(nodeval, c_idx, v_idx)

    out = _tc_reduce(_tc_finish_body, part2, maskf, np_)
    return out[:n_v]
